# baseline (speedup 1.0000x reference)
import math
import functools

import jax
import jax.numpy as jnp
from jax.experimental import pallas as pl
from jax.experimental.pallas import tpu as pltpu

_BLOCKSIZE = 32
_BATCH_TILE = 8


def _attn_kernel(x_ref, wqkv_ref, bqkv_ref, wproj_ref, bproj_ref, o_ref,
                 *, num_heads, head_dim, blocksize, batch_tile):
    N = x_ref.shape[1]
    C = num_heads * head_dim

    row = jax.lax.broadcasted_iota(jnp.int32, (N, N), 0) // blocksize
    col = jax.lax.broadcasted_iota(jnp.int32, (N, N), 1) // blocksize
    blk = (row == col).astype(jnp.float32)
    e_mask = (1.0 + (math.e - 1.0) * blk).astype(jnp.bfloat16)

    ones_pad = jnp.ones((N, head_dim), jnp.bfloat16)

    for e in range(batch_tile):
        x = x_ref[e].astype(jnp.bfloat16)
        qkv = jnp.dot(x, wqkv_ref[...], preferred_element_type=jnp.float32)
        qkv = (qkv + bqkv_ref[...]).astype(jnp.bfloat16)

        outs = []
        for h in range(num_heads):
            lo = h * head_dim
            hi = lo + head_dim
            q = qkv[:, lo:hi]
            k = qkv[:, C + lo:C + hi]
            v = qkv[:, 2 * C + lo:2 * C + hi]

            s = jax.lax.dot_general(
                q, k, (((1,), (1,)), ((), ())),
                preferred_element_type=jnp.float32)
            p = jnp.exp(s).astype(jnp.bfloat16) * e_mask
            v_aug = jnp.concatenate([v, ones_pad], axis=1)
            o_full = jnp.dot(p, v_aug, preferred_element_type=jnp.float32)
            o_h = (o_full[:, :head_dim]
                   * pl.reciprocal(o_full[:, head_dim:2 * head_dim]))
            outs.append(o_h.astype(jnp.bfloat16))

        attn = jnp.concatenate(outs, axis=1)
        out = jnp.dot(attn, wproj_ref[...],
                      preferred_element_type=jnp.float32)
        o_ref[e] = out + bproj_ref[...]


def kernel(x, w_qkv, b_qkv, w_proj, b_proj):
    B, N, C = x.shape
    num_heads = 12
    head_dim = C // num_heads
    scale = 1.0 / math.sqrt(head_dim)

    scale_vec = jnp.concatenate(
        [jnp.full((C,), scale, jnp.float32),
         jnp.ones((2 * C,), jnp.float32)])
    wqkv_bf = (w_qkv * scale_vec[None, :]).astype(jnp.bfloat16)
    bqkv_s = b_qkv * scale_vec[None, :]
    wproj_bf = w_proj.astype(jnp.bfloat16)

    bt = _BATCH_TILE
    body = functools.partial(
        _attn_kernel, num_heads=num_heads, head_dim=head_dim,
        blocksize=_BLOCKSIZE, batch_tile=bt)

    return pl.pallas_call(
        body,
        out_shape=jax.ShapeDtypeStruct((B, N, C), jnp.float32),
        grid=(B // bt,),
        in_specs=[
            pl.BlockSpec((bt, N, C), lambda b: (b, 0, 0)),
            pl.BlockSpec((C, 3 * C), lambda b: (0, 0)),
            pl.BlockSpec((1, 3 * C), lambda b: (0, 0)),
            pl.BlockSpec((C, C), lambda b: (0, 0)),
            pl.BlockSpec((1, C), lambda b: (0, 0)),
        ],
        out_specs=pl.BlockSpec((bt, N, C), lambda b: (b, 0, 0)),
        compiler_params=pltpu.CompilerParams(
            dimension_semantics=("parallel",)),
    )(x, wqkv_bf, bqkv_s, wproj_bf, b_proj)


# speedup vs baseline: 1.2428x; 1.2428x over previous
import math
import functools

import jax
import jax.numpy as jnp
from jax.experimental import pallas as pl
from jax.experimental.pallas import tpu as pltpu

_BLOCKSIZE = 32
_BATCH_TILE = 4


def _attn_kernel(x_ref, wqkv_ref, bqkv_ref, wproj_ref, bproj_ref, o_ref,
                 *, num_heads, head_dim, blocksize, batch_tile):
    N = x_ref.shape[1]
    C = num_heads * head_dim

    row = jax.lax.broadcasted_iota(jnp.int32, (N, N), 0) // blocksize
    col = jax.lax.broadcasted_iota(jnp.int32, (N, N), 1) // blocksize
    blk = (row == col).astype(jnp.float32)
    e_mask = (1.0 + (math.e - 1.0) * blk).astype(jnp.bfloat16)

    ones_pad = jnp.ones((N, head_dim), jnp.bfloat16)

    x_all = x_ref[...].astype(jnp.bfloat16).reshape(batch_tile * N, C)
    qkv_all = jnp.dot(x_all, wqkv_ref[...],
                      preferred_element_type=jnp.float32)
    qkv_all = (qkv_all + bqkv_ref[...]).astype(jnp.bfloat16)

    attns = []
    for e in range(batch_tile):
        qkv = qkv_all[e * N:(e + 1) * N]

        outs = []
        for h in range(num_heads):
            lo = h * head_dim
            hi = lo + head_dim
            q = qkv[:, lo:hi]
            k = qkv[:, C + lo:C + hi]
            v = qkv[:, 2 * C + lo:2 * C + hi]

            s = jax.lax.dot_general(
                q, k, (((1,), (1,)), ((), ())),
                preferred_element_type=jnp.float32)
            p = jnp.exp(s).astype(jnp.bfloat16) * e_mask
            v_aug = jnp.concatenate([v, ones_pad], axis=1)
            o_full = jnp.dot(p, v_aug, preferred_element_type=jnp.float32)
            o_h = (o_full[:, :head_dim]
                   * pl.reciprocal(o_full[:, head_dim:2 * head_dim]))
            outs.append(o_h.astype(jnp.bfloat16))

        attns.append(jnp.concatenate(outs, axis=1))

    attn_all = jnp.concatenate(attns, axis=0)
    out_all = jnp.dot(attn_all, wproj_ref[...],
                      preferred_element_type=jnp.float32)
    out_all = out_all + bproj_ref[...]
    o_ref[...] = out_all.reshape(batch_tile, N, C)


def kernel(x, w_qkv, b_qkv, w_proj, b_proj):
    B, N, C = x.shape
    num_heads = 12
    head_dim = C // num_heads
    scale = 1.0 / math.sqrt(head_dim)

    scale_vec = jnp.concatenate(
        [jnp.full((C,), scale, jnp.float32),
         jnp.ones((2 * C,), jnp.float32)])
    wqkv_bf = (w_qkv * scale_vec[None, :]).astype(jnp.bfloat16)
    bqkv_s = b_qkv * scale_vec[None, :]
    wproj_bf = w_proj.astype(jnp.bfloat16)

    bt = _BATCH_TILE
    body = functools.partial(
        _attn_kernel, num_heads=num_heads, head_dim=head_dim,
        blocksize=_BLOCKSIZE, batch_tile=bt)

    return pl.pallas_call(
        body,
        out_shape=jax.ShapeDtypeStruct((B, N, C), jnp.float32),
        grid=(B // bt,),
        in_specs=[
            pl.BlockSpec((bt, N, C), lambda b: (b, 0, 0)),
            pl.BlockSpec((C, 3 * C), lambda b: (0, 0)),
            pl.BlockSpec((1, 3 * C), lambda b: (0, 0)),
            pl.BlockSpec((C, C), lambda b: (0, 0)),
            pl.BlockSpec((1, C), lambda b: (0, 0)),
        ],
        out_specs=pl.BlockSpec((bt, N, C), lambda b: (b, 0, 0)),
        compiler_params=pltpu.CompilerParams(
            dimension_semantics=("parallel",)),
    )(x, wqkv_bf, bqkv_s, wproj_bf, b_proj)


# speedup vs baseline: 1.2765x; 1.0271x over previous
import math
import functools

import jax
import jax.numpy as jnp
from jax.experimental import pallas as pl
from jax.experimental.pallas import tpu as pltpu

_BLOCKSIZE = 32
_BATCH_TILE = 4


def _attn_kernel(x_ref, wqkv_ref, bqkv_ref, wproj_ref, bproj_ref, o_ref,
                 *, num_heads, head_dim, blocksize, batch_tile):
    N = x_ref.shape[1]
    C = num_heads * head_dim

    row = jax.lax.broadcasted_iota(jnp.int32, (N, N), 0) // blocksize
    col = jax.lax.broadcasted_iota(jnp.int32, (N, N), 1) // blocksize
    blk = (row == col).astype(jnp.float32)
    e_mask = (1.0 + (math.e - 1.0) * blk).astype(jnp.bfloat16)

    ones_pad = jnp.ones((N, head_dim), jnp.bfloat16)

    for e in range(batch_tile):
        x = x_ref[e].astype(jnp.bfloat16)
        qkv = jnp.dot(x, wqkv_ref[...], preferred_element_type=jnp.float32)
        qkv = (qkv + bqkv_ref[...]).astype(jnp.bfloat16)

        outs = []
        for h in range(num_heads):
            lo = h * head_dim
            hi = lo + head_dim
            q = qkv[:, lo:hi]
            k = qkv[:, C + lo:C + hi]
            v = qkv[:, 2 * C + lo:2 * C + hi]

            s = jax.lax.dot_general(
                q, k, (((1,), (1,)), ((), ())),
                preferred_element_type=jnp.float32)
            p = jnp.exp(s).astype(jnp.bfloat16) * e_mask
            v_aug = jnp.concatenate([v, ones_pad], axis=1)
            o_full = jnp.dot(p, v_aug, preferred_element_type=jnp.float32)
            o_h = (o_full[:, :head_dim]
                   * pl.reciprocal(o_full[:, head_dim:2 * head_dim]))
            outs.append(o_h.astype(jnp.bfloat16))

        attn = jnp.concatenate(outs, axis=1)
        out = jnp.dot(attn, wproj_ref[...],
                      preferred_element_type=jnp.float32)
        o_ref[e] = out + bproj_ref[...]


def kernel(x, w_qkv, b_qkv, w_proj, b_proj):
    B, N, C = x.shape
    num_heads = 12
    head_dim = C // num_heads
    scale = 1.0 / math.sqrt(head_dim)

    scale_vec = jnp.concatenate(
        [jnp.full((C,), scale, jnp.float32),
         jnp.ones((2 * C,), jnp.float32)])
    wqkv_bf = (w_qkv * scale_vec[None, :]).astype(jnp.bfloat16)
    bqkv_s = b_qkv * scale_vec[None, :]
    wproj_bf = w_proj.astype(jnp.bfloat16)

    bt = _BATCH_TILE
    body = functools.partial(
        _attn_kernel, num_heads=num_heads, head_dim=head_dim,
        blocksize=_BLOCKSIZE, batch_tile=bt)

    return pl.pallas_call(
        body,
        out_shape=jax.ShapeDtypeStruct((B, N, C), jnp.float32),
        grid=(B // bt,),
        in_specs=[
            pl.BlockSpec((bt, N, C), lambda b: (b, 0, 0)),
            pl.BlockSpec((C, 3 * C), lambda b: (0, 0)),
            pl.BlockSpec((1, 3 * C), lambda b: (0, 0)),
            pl.BlockSpec((C, C), lambda b: (0, 0)),
            pl.BlockSpec((1, C), lambda b: (0, 0)),
        ],
        out_specs=pl.BlockSpec((bt, N, C), lambda b: (b, 0, 0)),
        compiler_params=pltpu.CompilerParams(
            dimension_semantics=("parallel",)),
    )(x, wqkv_bf, bqkv_s, wproj_bf, b_proj)
